# baseline (device time: 155530 ns/iter reference)
import jax
import jax.numpy as jnp
from jax import lax
from jax.experimental import pallas as pl
from jax.experimental.pallas import tpu as pltpu

N_Z = 4


def kernel(O, Wo):
    B, S, Hs, D = O.shape
    K = Hs * D
    N = Wo.shape[1]
    s_per = S // N_Z
    n_hops = N_Z - 1

    x = O.transpose(0, 2, 3, 1).reshape(B, K, S)

    def body(
        x_ref, w_ref, out_ref, comm_ref, wb_ref, obuf_ref,
        send_sems, recv_sems, last_send_sems, last_recv_sems, out_sems,
    ):
        my_x = lax.axis_index("x")
        my_y = lax.axis_index("y")
        my_z = lax.axis_index("z")
        left = (my_z - 1) % N_Z
        right = (my_z + 1) % N_Z

        wb_ref[:, :] = w_ref[:, :].astype(jnp.bfloat16)

        def chunk_f32(c, b):
            xs = x_ref[b, :, pl.ds(c * s_per, s_per)].astype(jnp.bfloat16)
            return lax.dot_general(
                xs, wb_ref[:, :],
                (((0,), (0,)), ((), ())),
                preferred_element_type=jnp.float32,
            )

        rdmas = [
            [
                pltpu.make_async_remote_copy(
                    src_ref=comm_ref.at[h, b],
                    dst_ref=comm_ref.at[h + 1, b],
                    send_sem=send_sems.at[h, b],
                    recv_sem=recv_sems.at[h, b],
                    device_id=(my_x, my_y, right),
                    device_id_type=pl.DeviceIdType.MESH,
                )
                for b in range(B)
            ]
            for h in range(n_hops - 1)
        ]
        half = s_per // 2
        last_rdmas = [
            [
                pltpu.make_async_remote_copy(
                    src_ref=comm_ref.at[n_hops - 1, b, pl.ds(i * half, half)],
                    dst_ref=comm_ref.at[n_hops, b, pl.ds(i * half, half)],
                    send_sem=last_send_sems.at[b, i],
                    recv_sem=last_recv_sems.at[b, i],
                    device_id=(my_x, my_y, right),
                    device_id_type=pl.DeviceIdType.MESH,
                )
                for i in range(2)
            ]
            for b in range(B)
        ]

        c0 = (my_z - 1) % N_Z
        for b in range(B):
            comm_ref[0, b, :, :] = chunk_f32(c0, b).astype(jnp.bfloat16)

        barrier_sem = pltpu.get_barrier_semaphore()
        for nbr in (left, right):
            pl.semaphore_signal(
                barrier_sem, inc=1,
                device_id=(my_x, my_y, nbr),
                device_id_type=pl.DeviceIdType.MESH,
            )
        pl.semaphore_wait(barrier_sem, 2)

        for b in range(B):
            rdmas[0][b].start()

        for h in range(n_hops - 1):
            c = (my_z - 2 - h) % N_Z
            for b in range(B):
                t = chunk_f32(c, b).astype(jnp.bfloat16)
                rdmas[h][b].wait()
                comm_ref[h + 1, b, :, :] = comm_ref[h + 1, b, :, :] + t
                if h < n_hops - 2:
                    rdmas[h + 1][b].start()
                else:
                    last_rdmas[b][0].start()
                    last_rdmas[b][1].start()

        for b in range(B):
            t = chunk_f32(my_z, b)
            for i in range(2):
                sl = pl.ds(i * half, half)
                last_rdmas[b][i].wait()
                obuf_ref[b, sl, :] = (
                    t[i * half:(i + 1) * half, :]
                    + comm_ref[n_hops, b, sl, :].astype(jnp.float32)
                )
                pltpu.make_async_copy(
                    obuf_ref.at[b, sl], out_ref.at[b, sl], out_sems.at[b, i]
                ).start()

        for b in range(B):
            for i in range(2):
                sl = pl.ds(i * half, half)
                pltpu.make_async_copy(
                    obuf_ref.at[b, sl], out_ref.at[b, sl], out_sems.at[b, i]
                ).wait()

    return pl.pallas_call(
        body,
        out_shape=jax.ShapeDtypeStruct((B, s_per, N), jnp.float32),
        in_specs=[
            pl.BlockSpec(memory_space=pltpu.VMEM),
            pl.BlockSpec(memory_space=pltpu.VMEM),
        ],
        out_specs=pl.BlockSpec(memory_space=pl.ANY),
        scratch_shapes=[
            pltpu.VMEM((N_Z, B, s_per, N), jnp.bfloat16),
            pltpu.VMEM((K, N), jnp.bfloat16),
            pltpu.VMEM((B, s_per, N), jnp.float32),
            pltpu.SemaphoreType.DMA((N_Z - 2, B)),
            pltpu.SemaphoreType.DMA((N_Z - 2, B)),
            pltpu.SemaphoreType.DMA((B, 2)),
            pltpu.SemaphoreType.DMA((B, 2)),
            pltpu.SemaphoreType.DMA((B, 2)),
        ],
        compiler_params=pltpu.CompilerParams(
            collective_id=0, vmem_limit_bytes=100 * 1024 * 1024
        ),
    )(x, Wo)


# device time: 153692 ns/iter; 1.0120x vs baseline; 1.0120x over previous
import jax
import jax.numpy as jnp
from jax import lax
from jax.experimental import pallas as pl
from jax.experimental.pallas import tpu as pltpu

N_Z = 4


def kernel(O, Wo):
    B, S, Hs, D = O.shape
    K = Hs * D
    N = Wo.shape[1]
    s_per = S // N_Z
    n_hops = N_Z - 1

    x = O.transpose(0, 2, 3, 1).reshape(B, K, S)

    def body(
        x_ref, w_ref, out_ref, comm_ref, wb_ref, obuf_ref,
        send_sems, recv_sems, last_send_sems, last_recv_sems, out_sems,
    ):
        my_x = lax.axis_index("x")
        my_y = lax.axis_index("y")
        my_z = lax.axis_index("z")
        left = (my_z - 1) % N_Z
        right = (my_z + 1) % N_Z

        barrier_sem = pltpu.get_barrier_semaphore()
        for nbr in (left, right):
            pl.semaphore_signal(
                barrier_sem, inc=1,
                device_id=(my_x, my_y, nbr),
                device_id_type=pl.DeviceIdType.MESH,
            )
        pl.semaphore_wait(barrier_sem, 2)

        wb_ref[:, :] = w_ref[:, :].astype(jnp.bfloat16)

        def chunk_f32(c, b):
            xs = x_ref[b, :, pl.ds(c * s_per, s_per)].astype(jnp.bfloat16)
            return lax.dot_general(
                xs, wb_ref[:, :],
                (((0,), (0,)), ((), ())),
                preferred_element_type=jnp.float32,
            )

        rdmas = [
            [
                pltpu.make_async_remote_copy(
                    src_ref=comm_ref.at[h, b],
                    dst_ref=comm_ref.at[h + 1, b],
                    send_sem=send_sems.at[h, b],
                    recv_sem=recv_sems.at[h, b],
                    device_id=(my_x, my_y, right),
                    device_id_type=pl.DeviceIdType.MESH,
                )
                for b in range(B)
            ]
            for h in range(n_hops - 1)
        ]
        half = s_per // 2
        last_rdmas = [
            [
                pltpu.make_async_remote_copy(
                    src_ref=comm_ref.at[n_hops - 1, b, pl.ds(i * half, half)],
                    dst_ref=comm_ref.at[n_hops, b, pl.ds(i * half, half)],
                    send_sem=last_send_sems.at[b, i],
                    recv_sem=last_recv_sems.at[b, i],
                    device_id=(my_x, my_y, right),
                    device_id_type=pl.DeviceIdType.MESH,
                )
                for i in range(2)
            ]
            for b in range(B)
        ]

        c0 = (my_z - 1) % N_Z
        for b in range(B):
            comm_ref[0, b, :, :] = chunk_f32(c0, b).astype(jnp.bfloat16)
            rdmas[0][b].start()

        for h in range(n_hops - 1):
            c = (my_z - 2 - h) % N_Z
            for b in range(B):
                t = chunk_f32(c, b).astype(jnp.bfloat16)
                rdmas[h][b].wait()
                comm_ref[h + 1, b, :, :] = comm_ref[h + 1, b, :, :] + t
                if h < n_hops - 2:
                    rdmas[h + 1][b].start()
                else:
                    last_rdmas[b][0].start()
                    last_rdmas[b][1].start()

        for b in range(B):
            t = chunk_f32(my_z, b)
            for i in range(2):
                sl = pl.ds(i * half, half)
                last_rdmas[b][i].wait()
                obuf_ref[b, sl, :] = (
                    t[i * half:(i + 1) * half, :]
                    + comm_ref[n_hops, b, sl, :].astype(jnp.float32)
                )
                pltpu.make_async_copy(
                    obuf_ref.at[b, sl], out_ref.at[b, sl], out_sems.at[b, i]
                ).start()

        for b in range(B):
            for i in range(2):
                sl = pl.ds(i * half, half)
                pltpu.make_async_copy(
                    obuf_ref.at[b, sl], out_ref.at[b, sl], out_sems.at[b, i]
                ).wait()

    return pl.pallas_call(
        body,
        out_shape=jax.ShapeDtypeStruct((B, s_per, N), jnp.float32),
        in_specs=[
            pl.BlockSpec(memory_space=pltpu.VMEM),
            pl.BlockSpec(memory_space=pltpu.VMEM),
        ],
        out_specs=pl.BlockSpec(memory_space=pl.ANY),
        scratch_shapes=[
            pltpu.VMEM((N_Z, B, s_per, N), jnp.bfloat16),
            pltpu.VMEM((K, N), jnp.bfloat16),
            pltpu.VMEM((B, s_per, N), jnp.float32),
            pltpu.SemaphoreType.DMA((N_Z - 2, B)),
            pltpu.SemaphoreType.DMA((N_Z - 2, B)),
            pltpu.SemaphoreType.DMA((B, 2)),
            pltpu.SemaphoreType.DMA((B, 2)),
            pltpu.SemaphoreType.DMA((B, 2)),
        ],
        compiler_params=pltpu.CompilerParams(
            collective_id=0, vmem_limit_bytes=100 * 1024 * 1024
        ),
    )(x, Wo)
